# Initial kernel scaffold; baseline (speedup 1.0000x reference)
import functools

import jax
import jax.numpy as jnp
from jax import lax
from jax.experimental import pallas as pl
from jax.experimental.pallas import tpu as pltpu
from jax.experimental.pallas import tpu_sc as plsc

NUM_TABLES = 26
VOCAB = 100000
DIM = 32
BATCH = 4096
L = 20

NUM_WORKERS = 32
BAGS_PER_CHUNK = BATCH // NUM_WORKERS
ROWS_PER_CHUNK = BAGS_PER_CHUNK * L
GATHER_W = 128
N_GATHERS = ROWS_PER_CHUNK // GATHER_W
HALF = 16


def _sc_embedding_bag(table_flat, idx_flat):
    mesh = plsc.VectorSubcoreMesh(core_axis_name="c", subcore_axis_name="s")

    @functools.partial(
        pl.kernel,
        out_type=jax.ShapeDtypeStruct((BATCH, NUM_TABLES * DIM), jnp.float32),
        mesh=mesh,
        scratch_types=[
            pltpu.VMEM((ROWS_PER_CHUNK,), jnp.int32),
            pltpu.VMEM((ROWS_PER_CHUNK, DIM), jnp.float32),
            pltpu.VMEM((BAGS_PER_CHUNK, DIM), jnp.float32),
            pltpu.SemaphoreType.DMA,
        ],
    )
    def k(table_hbm, idx_hbm, out_hbm, idx_v, rows_v, out_v, sem):
        wid = lax.axis_index("s") * 2 + lax.axis_index("c")
        b0 = wid * BAGS_PER_CHUNK

        @pl.loop(0, NUM_TABLES)
        def _table(t):
            src = (t * BATCH + b0) * L
            pltpu.sync_copy(idx_hbm.at[pl.ds(src, ROWS_PER_CHUNK)], idx_v)
            for j in range(N_GATHERS):
                pltpu.async_copy(
                    table_hbm.at[idx_v.at[pl.ds(j * GATHER_W, GATHER_W)]],
                    rows_v.at[pl.ds(j * GATHER_W, GATHER_W)],
                    sem,
                )
            for j in range(N_GATHERS):
                pltpu.make_async_copy(
                    table_hbm.at[idx_v.at[pl.ds(j * GATHER_W, GATHER_W)]],
                    rows_v.at[pl.ds(j * GATHER_W, GATHER_W)],
                    sem,
                ).wait()

            @pl.loop(0, BAGS_PER_CHUNK)
            def _bag(i):
                r = i * L
                lo = rows_v[r, pl.ds(0, HALF)]
                hi = rows_v[r, pl.ds(HALF, HALF)]
                for l in range(1, L):
                    lo = lo + rows_v[r + l, pl.ds(0, HALF)]
                    hi = hi + rows_v[r + l, pl.ds(HALF, HALF)]
                out_v[i, pl.ds(0, HALF)] = lo
                out_v[i, pl.ds(HALF, HALF)] = hi

            pltpu.sync_copy(
                out_v,
                out_hbm.at[pl.ds(b0, BAGS_PER_CHUNK), pl.ds(t * DIM, DIM)],
            )

    return k(table_flat, idx_flat)


def kernel(indices, tables):
    offs = (jnp.arange(NUM_TABLES, dtype=jnp.int32) * VOCAB)[:, None, None]
    idx_flat = (indices.astype(jnp.int32) + offs).reshape(-1)
    table_flat = tables.reshape(NUM_TABLES * VOCAB, DIM)
    return _sc_embedding_bag(table_flat, idx_flat)


# speedup vs baseline: 6.8233x; 6.8233x over previous
import functools

import jax
import jax.numpy as jnp
from jax import lax
from jax.experimental import pallas as pl
from jax.experimental.pallas import tpu as pltpu
from jax.experimental.pallas import tpu_sc as plsc

NUM_TABLES = 26
VOCAB = 100000
DIM = 32
BATCH = 4096
L = 20

NUM_WORKERS = 32
BLOCK_BAGS = 64
BLOCKS_PER_WORKER = BATCH // (NUM_WORKERS * BLOCK_BAGS)
ROWS_PER_CHUNK = BLOCK_BAGS * L
GATHER_W = 128
N_GATHERS = ROWS_PER_CHUNK // GATHER_W
HALF = 16


def _sc_embedding_bag(table_flat, idx_flat):
    mesh = plsc.VectorSubcoreMesh(core_axis_name="c", subcore_axis_name="s")

    @functools.partial(
        pl.kernel,
        out_type=jax.ShapeDtypeStruct((BATCH, NUM_TABLES * DIM), jnp.float32),
        mesh=mesh,
        scratch_types=[
            pltpu.VMEM((ROWS_PER_CHUNK,), jnp.int32),
            pltpu.VMEM((ROWS_PER_CHUNK, DIM), jnp.float32),
            pltpu.VMEM((BLOCK_BAGS, NUM_TABLES * DIM), jnp.float32),
            pltpu.SemaphoreType.DMA,
        ],
        compiler_params=pltpu.CompilerParams(use_tc_tiling_on_sc=False),
    )
    def k(table_hbm, idx_hbm, out_hbm, idx_v, rows_v, out_v, sem):
        i32 = jnp.int32
        wid = lax.axis_index("s") * i32(2) + lax.axis_index("c")

        for kb in range(BLOCKS_PER_WORKER):
            b0 = wid * i32(NUM_WORKERS * BLOCK_BAGS // 16) + i32(kb * BLOCK_BAGS)

            def _table(t, _):
                src = (t * i32(BATCH) + b0) * i32(L)
                pltpu.sync_copy(idx_hbm.at[pl.ds(src, ROWS_PER_CHUNK)], idx_v)
                for j in range(N_GATHERS):
                    pltpu.async_copy(
                        table_hbm.at[idx_v.at[pl.ds(j * GATHER_W, GATHER_W)]],
                        rows_v.at[pl.ds(j * GATHER_W, GATHER_W)],
                        sem,
                    )
                for j in range(N_GATHERS):
                    pltpu.make_async_copy(
                        table_hbm.at[idx_v.at[pl.ds(j * GATHER_W, GATHER_W)]],
                        rows_v.at[pl.ds(j * GATHER_W, GATHER_W)],
                        sem,
                    ).wait()

                col = t * i32(DIM)

                def _bag(i, _):
                    r = i * i32(L)
                    lo = rows_v[r, pl.ds(0, HALF)]
                    hi = rows_v[r, pl.ds(HALF, HALF)]
                    for l in range(1, L):
                        lo = lo + rows_v[r + i32(l), pl.ds(0, HALF)]
                        hi = hi + rows_v[r + i32(l), pl.ds(HALF, HALF)]
                    out_v[i, pl.ds(col, HALF)] = lo
                    out_v[i, pl.ds(col + i32(HALF), HALF)] = hi
                    return _

                lax.fori_loop(i32(0), i32(BLOCK_BAGS), _bag, None)
                return _

            lax.fori_loop(i32(0), i32(NUM_TABLES), _table, None)

            pltpu.sync_copy(out_v, out_hbm.at[pl.ds(b0, BLOCK_BAGS), :])

    return k(table_flat, idx_flat)


def kernel(indices, tables):
    offs = (jnp.arange(NUM_TABLES, dtype=jnp.int32) * VOCAB)[:, None, None]
    idx_flat = (indices.astype(jnp.int32) + offs).reshape(-1)
    table_flat = tables.reshape(NUM_TABLES * VOCAB, DIM)
    return _sc_embedding_bag(table_flat, idx_flat)


# speedup vs baseline: 7.3157x; 1.0722x over previous
import functools

import jax
import jax.numpy as jnp
from jax import lax
from jax.experimental import pallas as pl
from jax.experimental.pallas import tpu as pltpu
from jax.experimental.pallas import tpu_sc as plsc

NUM_TABLES = 26
VOCAB = 100000
DIM = 32
BATCH = 4096
L = 20

NUM_WORKERS = 32
BLOCK_BAGS = 64
BAGS_PER_WORKER = BATCH // NUM_WORKERS
BLOCKS_PER_WORKER = BAGS_PER_WORKER // BLOCK_BAGS
HALF = 16


def _sc_embedding_bag(tables, idx3):
    mesh = plsc.VectorSubcoreMesh(core_axis_name="c", subcore_axis_name="s")

    @functools.partial(
        pl.kernel,
        out_type=jax.ShapeDtypeStruct((BATCH, NUM_TABLES * DIM), jnp.float32),
        mesh=mesh,
        scratch_types=[
            pltpu.VMEM((L, BLOCK_BAGS), jnp.int32),
            pltpu.VMEM((L * BLOCK_BAGS, DIM), jnp.float32),
            pltpu.VMEM((BLOCK_BAGS, NUM_TABLES * DIM), jnp.float32),
            pltpu.SemaphoreType.DMA,
        ],
        compiler_params=pltpu.CompilerParams(use_tc_tiling_on_sc=False),
    )
    def k(table_hbm, idx_hbm, out_hbm, idx_v, rows_v, out_v, sem):
        i32 = jnp.int32
        wid = lax.axis_index("s") * i32(2) + lax.axis_index("c")

        for kb in range(BLOCKS_PER_WORKER):
            b0 = wid * i32(BAGS_PER_WORKER) + i32(kb * BLOCK_BAGS)

            def _table(t, _):
                pltpu.sync_copy(
                    idx_hbm.at[t, :, pl.ds(b0, BLOCK_BAGS)], idx_v
                )
                tab_t = table_hbm.at[t]
                for j in range(L):
                    pltpu.async_copy(
                        tab_t.at[idx_v.at[i32(j)]],
                        rows_v.at[pl.ds(j * BLOCK_BAGS, BLOCK_BAGS)],
                        sem,
                    )
                for j in range(L):
                    pltpu.make_async_copy(
                        tab_t.at[idx_v.at[i32(j)]],
                        rows_v.at[pl.ds(j * BLOCK_BAGS, BLOCK_BAGS)],
                        sem,
                    ).wait()

                col = t * i32(DIM)

                def _bag(i, _):
                    lo = rows_v[i, pl.ds(0, HALF)]
                    hi = rows_v[i, pl.ds(HALF, HALF)]
                    for l in range(1, L):
                        r = i + i32(l * BLOCK_BAGS)
                        lo = lo + rows_v[r, pl.ds(0, HALF)]
                        hi = hi + rows_v[r, pl.ds(HALF, HALF)]
                    out_v[i, pl.ds(col, HALF)] = lo
                    out_v[i, pl.ds(col + i32(HALF), HALF)] = hi
                    return _

                lax.fori_loop(i32(0), i32(BLOCK_BAGS), _bag, None)
                return _

            lax.fori_loop(i32(0), i32(NUM_TABLES), _table, None)

            pltpu.sync_copy(out_v, out_hbm.at[pl.ds(b0, BLOCK_BAGS), :])

    return k(tables, idx3)


def kernel(indices, tables):
    idx3 = jnp.transpose(indices, (0, 2, 1)).astype(jnp.int32)
    return _sc_embedding_bag(tables, idx3)


# speedup vs baseline: 9.0395x; 1.2356x over previous
import functools

import jax
import jax.numpy as jnp
from jax import lax
from jax.experimental import pallas as pl
from jax.experimental.pallas import tpu as pltpu
from jax.experimental.pallas import tpu_sc as plsc

NUM_TABLES = 26
VOCAB = 100000
DIM = 32
BATCH = 4096
L = 20

S = 25088
LSTEP = 1792
NSTEP = S // LSTEP + 1
LINES_T = NSTEP * LSTEP
TAIL0 = VOCAB - 32
OV3 = TAIL0 - LSTEP
B3 = 3 * S + (NSTEP - 2) * LSTEP

NUM_WORKERS = 32
BLOCK_BAGS = 64
BAGS_PER_WORKER = BATCH // NUM_WORKERS
BLOCKS_PER_WORKER = BAGS_PER_WORKER // BLOCK_BAGS
HALF = 16


def _tc_compact(t2):
    def body(in_ref, out_ref):
        i32 = jnp.int32
        c = pl.program_id(1)
        base = c * i32(LSTEP)

        @pl.when(c < NSTEP - 1)
        def _main():
            pieces = []
            for k in range(4):
                off = k * i32(S) + base
                if k == 3:
                    off = jnp.where(c == i32(NSTEP - 2), i32(OV3), off)
                off = pl.multiple_of(off, 128)
                pieces.append(in_ref[0, :, pl.ds(off, LSTEP)].T)
            out_ref[0] = jnp.concatenate(pieces, axis=1)

        @pl.when(c == NSTEP - 1)
        def _tail():
            xs = in_ref[0, :, pl.ds(TAIL0, 32)]
            val = jnp.concatenate(
                [xs.T, jnp.zeros((32, 96), jnp.float32)], axis=1
            )
            out_ref[0] = jnp.concatenate(
                [val, jnp.zeros((LSTEP - 32, 128), jnp.float32)], axis=0
            )

    return pl.pallas_call(
        body,
        grid=(NUM_TABLES, NSTEP),
        in_specs=[
            pl.BlockSpec((1, DIM, VOCAB), lambda t, c: (t, t * 0, t * 0)),
        ],
        out_specs=pl.BlockSpec(
            (1, LSTEP, 128), lambda t, c: (t, c, t * 0)
        ),
        out_shape=jax.ShapeDtypeStruct(
            (NUM_TABLES, LINES_T, 128), jnp.float32
        ),
        compiler_params=pltpu.CompilerParams(
            vmem_limit_bytes=48 * 1024 * 1024
        ),
    )(t2)


def _sc_embedding_bag(table_flat, idx3):
    mesh = plsc.VectorSubcoreMesh(core_axis_name="c", subcore_axis_name="s")

    @functools.partial(
        pl.kernel,
        out_type=jax.ShapeDtypeStruct((BATCH, NUM_TABLES * DIM), jnp.float32),
        mesh=mesh,
        scratch_types=[
            pltpu.VMEM((L, BLOCK_BAGS), jnp.int32),
            pltpu.VMEM((L * BLOCK_BAGS, DIM), jnp.float32),
            pltpu.VMEM((BLOCK_BAGS, NUM_TABLES * DIM), jnp.float32),
            pltpu.SemaphoreType.DMA,
        ],
        compiler_params=pltpu.CompilerParams(use_tc_tiling_on_sc=False),
    )
    def k(table_hbm, idx_hbm, out_hbm, idx_v, rows_v, out_v, sem):
        i32 = jnp.int32
        wid = lax.axis_index("s") * i32(2) + lax.axis_index("c")

        for kb in range(BLOCKS_PER_WORKER):
            b0 = wid * i32(BAGS_PER_WORKER) + i32(kb * BLOCK_BAGS)

            def _table(t, _):
                pltpu.sync_copy(
                    idx_hbm.at[t, :, pl.ds(b0, BLOCK_BAGS)], idx_v
                )
                for j in range(L):
                    pltpu.async_copy(
                        table_hbm.at[idx_v.at[i32(j)]],
                        rows_v.at[pl.ds(j * BLOCK_BAGS, BLOCK_BAGS)],
                        sem,
                    )
                for j in range(L):
                    pltpu.make_async_copy(
                        table_hbm.at[idx_v.at[i32(j)]],
                        rows_v.at[pl.ds(j * BLOCK_BAGS, BLOCK_BAGS)],
                        sem,
                    ).wait()

                col = t * i32(DIM)

                def _bag(i, _):
                    lo = rows_v[i, pl.ds(0, HALF)]
                    hi = rows_v[i, pl.ds(HALF, HALF)]
                    for l in range(1, L):
                        r = i + i32(l * BLOCK_BAGS)
                        lo = lo + rows_v[r, pl.ds(0, HALF)]
                        hi = hi + rows_v[r, pl.ds(HALF, HALF)]
                    out_v[i, pl.ds(col, HALF)] = lo
                    out_v[i, pl.ds(col + i32(HALF), HALF)] = hi
                    return _

                lax.fori_loop(i32(0), i32(BLOCK_BAGS), _bag, None)
                return _

            lax.fori_loop(i32(0), i32(NUM_TABLES), _table, None)

            pltpu.sync_copy(out_v, out_hbm.at[pl.ds(b0, BLOCK_BAGS), :])

    return k(table_flat, idx3)


def kernel(indices, tables):
    offs = (jnp.arange(NUM_TABLES, dtype=jnp.int32) * (LINES_T * 4))[
        :, None, None
    ]
    v = jnp.transpose(indices, (0, 2, 1)).astype(jnp.int32)
    g = jnp.where(
        v < 3 * S,
        (v % S) * 4 + v // S,
        jnp.where(
            v < B3,
            (v - 3 * S) * 4 + 3,
            (v - OV3 + (NSTEP - 2) * LSTEP) * 4
            + jnp.where(v < TAIL0, 3, 0),
        ),
    )
    idx3 = g + offs
    t2 = jnp.transpose(tables, (0, 2, 1))
    packed = _tc_compact(t2)
    table_flat = packed.reshape(NUM_TABLES * LINES_T * 4, DIM)
    return _sc_embedding_bag(table_flat, idx3)


# speedup vs baseline: 15.0425x; 1.6641x over previous
import functools

import jax
import jax.numpy as jnp
from jax import lax
from jax.experimental import pallas as pl
from jax.experimental.pallas import tpu as pltpu
from jax.experimental.pallas import tpu_sc as plsc

NUM_TABLES = 26
VOCAB = 100000
DIM = 32
BATCH = 4096
L = 20

S = 25088
LSTEP = 3584
NSTEP = S // LSTEP + 1
LINES_T = NSTEP * LSTEP
TAIL0 = VOCAB - 32
OV3 = TAIL0 - LSTEP
B3 = 3 * S + (NSTEP - 2) * LSTEP

NUM_WORKERS = 32
BLOCK_BAGS = 64
BAGS_PER_WORKER = BATCH // NUM_WORKERS
BLOCKS_PER_WORKER = BAGS_PER_WORKER // BLOCK_BAGS
HALF = 16


def _tc_compact(t2):
    def body(in_ref, out_ref):
        i32 = jnp.int32
        c = pl.program_id(1)
        base = c * i32(LSTEP)

        @pl.when(c < NSTEP - 1)
        def _main():
            pieces = []
            for k in range(4):
                off = k * i32(S) + base
                if k == 3:
                    off = jnp.where(c == i32(NSTEP - 2), i32(OV3), off)
                off = pl.multiple_of(off, 128)
                pieces.append(in_ref[0, :, pl.ds(off, LSTEP)])
            out_ref[0] = jnp.concatenate(pieces, axis=0).T

        @pl.when(c == NSTEP - 1)
        def _tail():
            xs = in_ref[0, :, pl.ds(TAIL0, 32)]
            val = jnp.concatenate(
                [xs.T, jnp.zeros((32, 96), jnp.float32)], axis=1
            )
            out_ref[0] = jnp.concatenate(
                [val, jnp.zeros((LSTEP - 32, 128), jnp.float32)], axis=0
            )

    return pl.pallas_call(
        body,
        grid=(NUM_TABLES, NSTEP),
        in_specs=[
            pl.BlockSpec((1, DIM, VOCAB), lambda t, c: (t, t * 0, t * 0)),
        ],
        out_specs=pl.BlockSpec(
            (1, LSTEP, 128), lambda t, c: (t, c, t * 0)
        ),
        out_shape=jax.ShapeDtypeStruct(
            (NUM_TABLES, LINES_T, 128), jnp.float32
        ),
        compiler_params=pltpu.CompilerParams(
            vmem_limit_bytes=48 * 1024 * 1024
        ),
    )(t2)


def _sc_embedding_bag(table_flat, idx3):
    mesh = plsc.VectorSubcoreMesh(core_axis_name="c", subcore_axis_name="s")

    @functools.partial(
        pl.kernel,
        out_type=jax.ShapeDtypeStruct((BATCH, NUM_TABLES * DIM), jnp.float32),
        mesh=mesh,
        scratch_types=[
            pltpu.VMEM((L, BLOCK_BAGS), jnp.int32),
            pltpu.VMEM((L * BLOCK_BAGS, DIM), jnp.float32),
            pltpu.VMEM((BLOCK_BAGS, NUM_TABLES * DIM), jnp.float32),
            pltpu.SemaphoreType.DMA,
        ],
        compiler_params=pltpu.CompilerParams(use_tc_tiling_on_sc=False),
    )
    def k(table_hbm, idx_hbm, out_hbm, idx_v, rows_v, out_v, sem):
        i32 = jnp.int32
        wid = lax.axis_index("s") * i32(2) + lax.axis_index("c")

        for kb in range(BLOCKS_PER_WORKER):
            b0 = wid * i32(BAGS_PER_WORKER) + i32(kb * BLOCK_BAGS)

            def _table(t, _):
                pltpu.sync_copy(
                    idx_hbm.at[t, :, pl.ds(b0, BLOCK_BAGS)], idx_v
                )
                for j in range(L):
                    pltpu.async_copy(
                        table_hbm.at[idx_v.at[i32(j)]],
                        rows_v.at[pl.ds(j * BLOCK_BAGS, BLOCK_BAGS)],
                        sem,
                    )
                for j in range(L):
                    pltpu.make_async_copy(
                        table_hbm.at[idx_v.at[i32(j)]],
                        rows_v.at[pl.ds(j * BLOCK_BAGS, BLOCK_BAGS)],
                        sem,
                    ).wait()

                col = t * i32(DIM)

                def _bag(i, _):
                    lo = rows_v[i, pl.ds(0, HALF)]
                    hi = rows_v[i, pl.ds(HALF, HALF)]
                    for l in range(1, L):
                        r = i + i32(l * BLOCK_BAGS)
                        lo = lo + rows_v[r, pl.ds(0, HALF)]
                        hi = hi + rows_v[r, pl.ds(HALF, HALF)]
                    out_v[i, pl.ds(col, HALF)] = lo
                    out_v[i, pl.ds(col + i32(HALF), HALF)] = hi
                    return _

                lax.fori_loop(i32(0), i32(BLOCK_BAGS), _bag, None)
                return _

            lax.fori_loop(i32(0), i32(NUM_TABLES), _table, None)

            pltpu.sync_copy(out_v, out_hbm.at[pl.ds(b0, BLOCK_BAGS), :])

    return k(table_flat, idx3)


def kernel(indices, tables):
    offs = (jnp.arange(NUM_TABLES, dtype=jnp.int32) * (LINES_T * 4))[
        :, None, None
    ]
    v = jnp.transpose(indices, (0, 2, 1)).astype(jnp.int32)
    g = jnp.where(
        v < 3 * S,
        (v % S) * 4 + v // S,
        jnp.where(
            v < B3,
            (v - 3 * S) * 4 + 3,
            (v - OV3 + (NSTEP - 2) * LSTEP) * 4
            + jnp.where(v < TAIL0, 3, 0),
        ),
    )
    idx3 = g + offs
    t2 = jnp.transpose(tables, (0, 2, 1))
    packed = _tc_compact(t2)
    table_flat = packed.reshape(NUM_TABLES * LINES_T * 4, DIM)
    return _sc_embedding_bag(table_flat, idx3)
